# baseline (speedup 1.0000x reference)
import functools

import jax
import jax.numpy as jnp
from jax import lax
from jax.experimental import pallas as pl
from jax.experimental.pallas import tpu as pltpu
from jax.experimental.pallas import tpu_sc as plsc

NUM_CORES = 2
NUM_SUBCORES = 16
NUM_WORKERS = NUM_CORES * NUM_SUBCORES
CHUNK = 128
NBUF = 5
AHEAD = 3


@jax.jit
def kernel(inputs, embedding):
    batch, hist = inputs.shape
    vocab, feat = embedding.shape
    total = batch * hist
    rows_per_worker = total // NUM_WORKERS
    nchunk = rows_per_worker // CHUNK

    idx3d = inputs.reshape(NUM_WORKERS, nchunk, CHUNK).astype(jnp.int32)

    mesh = plsc.VectorSubcoreMesh(
        core_axis_name="c",
        subcore_axis_name="s",
        num_cores=NUM_CORES,
        num_subcores=NUM_SUBCORES,
    )

    @functools.partial(
        pl.kernel,
        mesh=mesh,
        out_type=jax.ShapeDtypeStruct((total, feat), jnp.float32),
        scratch_types=[
            pltpu.VMEM((nchunk, CHUNK), jnp.int32),
            [pltpu.VMEM((CHUNK, feat), jnp.float32) for _ in range(NBUF)],
            [pltpu.SemaphoreType.DMA for _ in range(NBUF)],
            [pltpu.SemaphoreType.DMA for _ in range(NBUF)],
        ],
    )
    def gather_kernel(idx_hbm, table_hbm, out_hbm, idx_v, bufs, sem_g, sem_s):
        wid = lax.axis_index("s") * NUM_CORES + lax.axis_index("c")
        cbase = wid * nchunk

        pltpu.sync_copy(idx_hbm.at[wid], idx_v)

        def fire_gather(j, b):
            pltpu.async_copy(table_hbm.at[idx_v.at[j]], bufs[b], sem_g[b])

        def wait_gather(j, b):
            pltpu.make_async_copy(
                table_hbm.at[idx_v.at[j]], bufs[b], sem_g[b]
            ).wait()

        def fire_scatter(j, b):
            pltpu.async_copy(
                bufs[b], out_hbm.at[pl.ds((cbase + j) * CHUNK, CHUNK)], sem_s[b]
            )

        def wait_scatter(j, b):
            pltpu.make_async_copy(
                bufs[b], out_hbm.at[pl.ds((cbase + j) * CHUNK, CHUNK)], sem_s[b]
            ).wait()

        for b in range(AHEAD):
            fire_gather(b, b)

        @pl.loop(0, nchunk, step=NBUF)
        def _(g):
            for b in range(NBUF):
                j = g + b
                wait_gather(j, b)
                fire_scatter(j, b)
                f = j + AHEAD
                bf = (b + AHEAD) % NBUF

                @pl.when(f < nchunk)
                def _fire():
                    @pl.when(f >= NBUF)
                    def _drain():
                        wait_scatter(f - NBUF, bf)

                    fire_gather(f, bf)

        for jt in range(nchunk - NBUF, nchunk):
            wait_scatter(jt, jt % NBUF)

    out = gather_kernel(idx3d, embedding)
    return out.reshape(batch, hist, feat)


# speedup vs baseline: 1.7813x; 1.7813x over previous
import functools

import jax
import jax.numpy as jnp
from jax import lax
from jax.experimental import pallas as pl
from jax.experimental.pallas import tpu as pltpu
from jax.experimental.pallas import tpu_sc as plsc

NUM_CORES = 2
NUM_SUBCORES = 16
NUM_WORKERS = NUM_CORES * NUM_SUBCORES
PPB = 2
NBUF = 4
AHEAD = 2


@jax.jit
def kernel(inputs, embedding):
    batch, hist = inputs.shape
    vocab, feat = embedding.shape
    bpw = batch // NUM_WORKERS
    nchunk = bpw // PPB

    idx = inputs.astype(jnp.int32)

    mesh = plsc.VectorSubcoreMesh(
        core_axis_name="c",
        subcore_axis_name="s",
        num_cores=NUM_CORES,
        num_subcores=NUM_SUBCORES,
    )

    @functools.partial(
        pl.kernel,
        mesh=mesh,
        out_type=jax.ShapeDtypeStruct((batch, hist, feat), jnp.float32),
        scratch_types=[
            pltpu.VMEM((bpw, hist), jnp.int32),
            [pltpu.VMEM((PPB, hist, feat), jnp.float32) for _ in range(NBUF)],
            [pltpu.SemaphoreType.DMA for _ in range(NBUF)],
            [pltpu.SemaphoreType.DMA for _ in range(NBUF)],
        ],
    )
    def gather_kernel(idx_hbm, table_hbm, out_hbm, idx_v, bufs, sem_g, sem_s):
        wid = lax.axis_index("s") * NUM_CORES + lax.axis_index("c")
        base = wid * bpw

        pltpu.sync_copy(idx_hbm.at[pl.ds(base, bpw)], idx_v)

        def fire_gather(j, b):
            for p in range(PPB):
                pltpu.async_copy(
                    table_hbm.at[idx_v.at[j * PPB + p]], bufs[b].at[p], sem_g[b]
                )

        def wait_gather(j, b):
            for p in range(PPB):
                pltpu.make_async_copy(
                    table_hbm.at[idx_v.at[j * PPB + p]], bufs[b].at[p], sem_g[b]
                ).wait()

        def fire_scatter(j, b):
            pltpu.async_copy(
                bufs[b], out_hbm.at[pl.ds(base + j * PPB, PPB)], sem_s[b]
            )

        def wait_scatter(j, b):
            pltpu.make_async_copy(
                bufs[b], out_hbm.at[pl.ds(base + j * PPB, PPB)], sem_s[b]
            ).wait()

        for b in range(AHEAD):
            fire_gather(b, b)

        @pl.loop(0, nchunk, step=NBUF)
        def _(g):
            for b in range(NBUF):
                j = g + b
                wait_gather(j, b)
                fire_scatter(j, b)
                f = j + AHEAD
                bf = (b + AHEAD) % NBUF

                @pl.when(f < nchunk)
                def _fire():
                    @pl.when(f >= NBUF)
                    def _drain():
                        wait_scatter(f - NBUF, bf)

                    fire_gather(f, bf)

        for jt in range(nchunk - NBUF, nchunk):
            wait_scatter(jt, jt % NBUF)

    return gather_kernel(idx, embedding)


# speedup vs baseline: 3.1381x; 1.7617x over previous
import functools

import jax
import jax.numpy as jnp
from jax import lax
from jax.experimental import pallas as pl
from jax.experimental.pallas import tpu as pltpu
from jax.experimental.pallas import tpu_sc as plsc

NUM_CORES = 2
NUM_SUBCORES = 16
NUM_WORKERS = NUM_CORES * NUM_SUBCORES
CHUNK = 128
NBUF = 5
AHEAD = 3


@jax.jit
def kernel(inputs, embedding):
    batch, hist = inputs.shape
    vocab, feat = embedding.shape
    total = batch * hist
    rows_per_worker = total // NUM_WORKERS
    nchunk = rows_per_worker // CHUNK

    idx3d = inputs.T.astype(jnp.int32).reshape(NUM_WORKERS, nchunk, CHUNK)

    mesh = plsc.VectorSubcoreMesh(
        core_axis_name="c",
        subcore_axis_name="s",
        num_cores=NUM_CORES,
        num_subcores=NUM_SUBCORES,
    )

    @functools.partial(
        pl.kernel,
        mesh=mesh,
        out_type=jax.ShapeDtypeStruct((total, feat), jnp.float32),
        scratch_types=[
            pltpu.VMEM((nchunk, CHUNK), jnp.int32),
            [pltpu.VMEM((CHUNK, feat), jnp.float32) for _ in range(NBUF)],
            [pltpu.SemaphoreType.DMA for _ in range(NBUF)],
            [pltpu.SemaphoreType.DMA for _ in range(NBUF)],
        ],
    )
    def gather_kernel(idx_hbm, table_hbm, out_hbm, idx_v, bufs, sem_g, sem_s):
        wid = lax.axis_index("s") * NUM_CORES + lax.axis_index("c")
        cbase = wid * nchunk

        pltpu.sync_copy(idx_hbm.at[wid], idx_v)

        def fire_gather(j, b):
            pltpu.async_copy(table_hbm.at[idx_v.at[j]], bufs[b], sem_g[b])

        def wait_gather(j, b):
            pltpu.make_async_copy(
                table_hbm.at[idx_v.at[j]], bufs[b], sem_g[b]
            ).wait()

        def fire_scatter(j, b):
            pltpu.async_copy(
                bufs[b], out_hbm.at[pl.ds((cbase + j) * CHUNK, CHUNK)], sem_s[b]
            )

        def wait_scatter(j, b):
            pltpu.make_async_copy(
                bufs[b], out_hbm.at[pl.ds((cbase + j) * CHUNK, CHUNK)], sem_s[b]
            ).wait()

        for b in range(AHEAD):
            fire_gather(b, b)

        @pl.loop(0, nchunk, step=NBUF)
        def _(g):
            for b in range(NBUF):
                j = g + b
                wait_gather(j, b)
                fire_scatter(j, b)
                f = j + AHEAD
                bf = (b + AHEAD) % NBUF

                @pl.when(f < nchunk)
                def _fire():
                    @pl.when(f >= NBUF)
                    def _drain():
                        wait_scatter(f - NBUF, bf)

                    fire_gather(f, bf)

        for jt in range(nchunk - NBUF, nchunk):
            wait_scatter(jt, jt % NBUF)

    out = gather_kernel(idx3d, embedding)
    return out.reshape(hist, batch, feat).transpose(1, 0, 2)
